# Initial kernel scaffold; baseline (speedup 1.0000x reference)
import functools

import jax
import jax.numpy as jnp
from jax import lax
from jax.experimental import pallas as pl
from jax.experimental.pallas import tpu as pltpu
from jax.experimental.pallas import tpu_sc as plsc

NNODE = 10000
NP = 10240
NEDGE = 320000
NCORE = 2
NSUB = 16
NWORK = NCORE * NSUB
ROWS_PER_TILE = NP // NSUB
F32 = jnp.float32

RBLK = 1024
TGRID = NP // RBLK


def _att_mat(att, heads, oc):
    a = att.reshape(heads * oc)
    head_of_row = jnp.arange(heads * oc) // oc
    head_of_col = jnp.arange(16) % heads
    return jnp.where(head_of_row[:, None] == head_of_col[None, :],
                     a[:, None], 0.0).astype(F32)


def _bcast_mat(heads, oc):
    head_of_col = jnp.arange(heads * oc) // oc
    return (jnp.arange(heads)[:, None] == head_of_col[None, :]).astype(F32)


def _tc_pre(x, W, Asrc, Adst):
    DH = W.shape[1]

    def body(x_ref, w_ref, as_ref, ad_ref, h_ref, ts_ref, td_ref, gm_ref, mscr):
        xv = x_ref[...]
        hm = jnp.dot(xv, w_ref[...], preferred_element_type=F32)
        h_ref[...] = hm
        ts = jnp.dot(hm, as_ref[...], preferred_element_type=F32)
        td = jnp.dot(hm, ad_ref[...], preferred_element_type=F32)
        ts_ref[...] = ts
        td_ref[...] = td

        @pl.when(pl.program_id(0) == 0)
        def _():
            mscr[...] = jnp.full((1, 16), -1e30, F32)

        mscr[...] = jnp.maximum(mscr[...], jnp.max(ts, axis=0, keepdims=True))
        gm_ref[...] = mscr[...]

    return pl.pallas_call(
        body,
        grid=(TGRID,),
        in_specs=[
            pl.BlockSpec((RBLK, x.shape[1]), lambda i: (i, 0)),
            pl.BlockSpec(W.shape, lambda i: (0, 0)),
            pl.BlockSpec((DH, 16), lambda i: (0, 0)),
            pl.BlockSpec((DH, 16), lambda i: (0, 0)),
        ],
        out_specs=[
            pl.BlockSpec((RBLK, DH), lambda i: (i, 0)),
            pl.BlockSpec((RBLK, 16), lambda i: (i, 0)),
            pl.BlockSpec((RBLK, 16), lambda i: (i, 0)),
            pl.BlockSpec((1, 16), lambda i: (0, 0)),
        ],
        out_shape=[
            jax.ShapeDtypeStruct((NP, DH), F32),
            jax.ShapeDtypeStruct((NP, 16), F32),
            jax.ShapeDtypeStruct((NP, 16), F32),
            jax.ShapeDtypeStruct((1, 16), F32),
        ],
        scratch_shapes=[pltpu.VMEM((1, 16), F32)],
    )(x, W, Asrc, Adst)


def _tc_mid(acc, bias, B, W, Asrc, Adst, dprev, hprev):
    DH = W.shape[1]
    ACCP = acc.shape[2]

    def body(acc_ref, b_ref, bm_ref, w_ref, as_ref, ad_ref,
             x_ref, h_ref, ts_ref, td_ref, gm_ref, mscr):
        a = acc_ref[0] + acc_ref[1]
        num = a[:, :dprev]
        den = a[:, dprev:dprev + hprev]
        dexp = jnp.dot(den, bm_ref[...], preferred_element_type=F32)
        xq = num / (dexp + 1e-16) + b_ref[...]
        xv = jnp.where(xq > 0, xq, jnp.exp(xq) - 1.0)
        x_ref[...] = xv
        hm = jnp.dot(xv, w_ref[...], preferred_element_type=F32)
        h_ref[...] = hm
        ts = jnp.dot(hm, as_ref[...], preferred_element_type=F32)
        td = jnp.dot(hm, ad_ref[...], preferred_element_type=F32)
        ts_ref[...] = ts
        td_ref[...] = td

        @pl.when(pl.program_id(0) == 0)
        def _():
            mscr[...] = jnp.full((1, 16), -1e30, F32)

        mscr[...] = jnp.maximum(mscr[...], jnp.max(ts, axis=0, keepdims=True))
        gm_ref[...] = mscr[...]

    return pl.pallas_call(
        body,
        grid=(TGRID,),
        in_specs=[
            pl.BlockSpec((2, RBLK, ACCP), lambda i: (0, i, 0)),
            pl.BlockSpec((1, dprev), lambda i: (0, 0)),
            pl.BlockSpec((hprev, dprev), lambda i: (0, 0)),
            pl.BlockSpec(W.shape, lambda i: (0, 0)),
            pl.BlockSpec((DH, 16), lambda i: (0, 0)),
            pl.BlockSpec((DH, 16), lambda i: (0, 0)),
        ],
        out_specs=[
            pl.BlockSpec((RBLK, dprev), lambda i: (i, 0)),
            pl.BlockSpec((RBLK, DH), lambda i: (i, 0)),
            pl.BlockSpec((RBLK, 16), lambda i: (i, 0)),
            pl.BlockSpec((RBLK, 16), lambda i: (i, 0)),
            pl.BlockSpec((1, 16), lambda i: (0, 0)),
        ],
        out_shape=[
            jax.ShapeDtypeStruct((NP, dprev), F32),
            jax.ShapeDtypeStruct((NP, DH), F32),
            jax.ShapeDtypeStruct((NP, 16), F32),
            jax.ShapeDtypeStruct((NP, 16), F32),
            jax.ShapeDtypeStruct((1, 16), F32),
        ],
        scratch_shapes=[pltpu.VMEM((1, 16), F32)],
    )(acc, bias, B, W, Asrc, Adst)


def _tc_post(acc, bias, B, dprev, hprev):
    ACCP = acc.shape[2]

    def body(acc_ref, b_ref, bm_ref, x_ref):
        a = acc_ref[0] + acc_ref[1]
        num = a[:, :dprev]
        den = a[:, dprev:dprev + hprev]
        dexp = jnp.dot(den, bm_ref[...], preferred_element_type=F32)
        xq = num / (dexp + 1e-16) + b_ref[...]
        x_ref[...] = jnp.where(xq > 0, xq, jnp.exp(xq) - 1.0)

    return pl.pallas_call(
        body,
        grid=(TGRID,),
        in_specs=[
            pl.BlockSpec((2, RBLK, ACCP), lambda i: (0, i, 0)),
            pl.BlockSpec((1, dprev), lambda i: (0, 0)),
            pl.BlockSpec((hprev, dprev), lambda i: (0, 0)),
        ],
        out_specs=pl.BlockSpec((RBLK, dprev), lambda i: (i, 0)),
        out_shape=jax.ShapeDtypeStruct((NP, dprev), F32),
    )(acc, bias, B)


def _sc_edge(h, tsrc, tdst, gmax, srcv, dstv, D, H, chunk=16):
    ACC = D + 16
    EP = srcv.shape[0]
    EPW = EP // NWORK
    nchunk = EPW // chunk
    nmul = D // 16
    mesh = plsc.VectorSubcoreMesh(core_axis_name="c", subcore_axis_name="s")

    def body(h_hbm, ts_hbm, td_hbm, gm_hbm, src_hbm, dst_hbm, out_hbm,
             srcb, dstb, tsb, tdb, hb, msgb, esc, gmv, zb, accs,
             sg1, sg2, sg3):
        cid = lax.axis_index("c")
        sid = lax.axis_index("s")
        wid = sid * NCORE + cid

        for r in range(16):
            for c in range(ACC // 16):
                zb[r, pl.ds(c * 16, 16)] = jnp.zeros((16,), F32)

        def zloop(j, carry):
            pltpu.sync_copy(zb, accs.at[pl.ds(sid * ROWS_PER_TILE + j * 16, 16)])
            return carry

        lax.fori_loop(0, ROWS_PER_TILE // 16, zloop, 0)

        pltpu.sync_copy(gm_hbm, gmv)
        base = wid * EPW
        pltpu.sync_copy(src_hbm.at[pl.ds(base, EPW)], srcb)
        pltpu.sync_copy(dst_hbm.at[pl.ds(base, EPW)], dstb)
        plsc.subcore_barrier()

        def chunk_body(t, carry):
            srciv = srcb[pl.ds(t * chunk, chunk)]
            dstiv = dstb[pl.ds(t * chunk, chunk)]
            c1 = pltpu.async_copy(ts_hbm.at[srciv], tsb, sg1)
            c2 = pltpu.async_copy(td_hbm.at[dstiv], tdb, sg2)
            c3 = pltpu.async_copy(h_hbm.at[srciv], hb, sg3)
            c1.wait()
            c2.wait()
            c3.wait()
            gm = gmv[...]
            for e in range(chunk):
                asv = tsb[e, :]
                adv = tdb[e, :]
                al = asv + adv
                al = jnp.maximum(al, 0.2 * al)
                sh = adv + gm
                sh = jnp.maximum(sh, 0.2 * sh)
                ev = jnp.exp(al - sh)
                msgb[e, pl.ds(D, 16)] = ev
                if H == 1:
                    for m in range(nmul):
                        msgb[e, pl.ds(m * 16, 16)] = hb[e, pl.ds(m * 16, 16)] * ev
                else:
                    esc[...] = ev
                    for m in range(nmul):
                        evh = plsc.load_gather(
                            esc, [jnp.full((16,), m % H, jnp.int32)])
                        msgb[e, pl.ds(m * 16, 16)] = hb[e, pl.ds(m * 16, 16)] * evh
            pltpu.sync_copy(msgb, accs.at[dstiv], add=True)
            return carry

        lax.fori_loop(0, nchunk, chunk_body, 0)
        plsc.subcore_barrier()
        pltpu.sync_copy(accs.at[pl.ds(sid * ROWS_PER_TILE, ROWS_PER_TILE)],
                        out_hbm.at[cid, pl.ds(sid * ROWS_PER_TILE, ROWS_PER_TILE)])

    kfn = pl.kernel(
        body,
        out_type=jax.ShapeDtypeStruct((NCORE, NP, ACC), F32),
        mesh=mesh,
        scratch_types=[
            pltpu.VMEM((EPW,), jnp.int32),
            pltpu.VMEM((EPW,), jnp.int32),
            pltpu.VMEM((chunk, 16), F32),
            pltpu.VMEM((chunk, 16), F32),
            pltpu.VMEM((chunk, D), F32),
            pltpu.VMEM((chunk, ACC), F32),
            pltpu.VMEM((16,), F32),
            pltpu.VMEM((16,), F32),
            pltpu.VMEM((16, ACC), F32),
            pltpu.VMEM_SHARED((NP, ACC), F32),
            pltpu.SemaphoreType.DMA,
            pltpu.SemaphoreType.DMA,
            pltpu.SemaphoreType.DMA,
        ],
    )
    return kfn(h, tsrc, tdst, gmax, srcv, dstv)


def kernel(x, edge_index, params):
    layers = params["layers"]
    cfg = [(8, 16)] * 4 + [(1, 32)]

    src = edge_index[0].astype(jnp.int32)
    dst = edge_index[1].astype(jnp.int32)
    loop = jnp.arange(NNODE, dtype=jnp.int32)
    etot = NEDGE + NNODE
    ep = ((etot + NWORK * 16 - 1) // (NWORK * 16)) * NWORK * 16
    pad = ep - etot
    srcv = jnp.concatenate([src, loop, jnp.zeros((pad,), jnp.int32)])
    dstv = jnp.concatenate([dst, loop, jnp.full((pad,), NNODE, jnp.int32)])
    xp = jnp.pad(x, ((0, NP - NNODE), (0, 0)))

    p0 = layers[0]
    h, ts, td, gm = _tc_pre(xp, p0["W"],
                            _att_mat(p0["att_src"], 8, 16),
                            _att_mat(p0["att_dst"], 8, 16))
    acc = _sc_edge(h, ts, td, gm.reshape(16), srcv, dstv, D=128, H=8)

    B8 = _bcast_mat(8, 16)
    reps = []
    for li in range(1, 5):
        p = layers[li]
        heads, oc = cfg[li]
        prev_bias = layers[li - 1]["bias"].reshape(1, -1).astype(F32)
        xl, h, ts, td, gm = _tc_mid(acc, prev_bias, B8, p["W"],
                                    _att_mat(p["att_src"], heads, oc),
                                    _att_mat(p["att_dst"], heads, oc),
                                    dprev=128, hprev=8)
        reps.append(xl[:NNODE])
        acc = _sc_edge(h, ts, td, gm.reshape(16), srcv, dstv,
                       D=heads * oc, H=heads)

    out = _tc_post(acc, layers[4]["bias"].reshape(1, -1).astype(F32),
                   jnp.ones((1, 32), F32), dprev=32, hprev=1)
    outn = out[:NNODE]
    reps.append(outn)
    return (outn, reps)


# speedup vs baseline: 56.2178x; 56.2178x over previous
import functools

import jax
import jax.numpy as jnp
from jax import lax
from jax.experimental import pallas as pl
from jax.experimental.pallas import tpu as pltpu
from jax.experimental.pallas import tpu_sc as plsc

NNODE = 10000
NP = 10240
NEDGE = 320000
NCORE = 2
NSUB = 16
NWORK = NCORE * NSUB
ROWS_PER_TILE = NP // NSUB
F32 = jnp.float32

RBLK = 1024
TGRID = NP // RBLK


def _att_mat(att, heads, oc):
    a = att.reshape(heads * oc)
    head_of_row = jnp.arange(heads * oc) // oc
    head_of_col = jnp.arange(16) % heads
    return jnp.where(head_of_row[:, None] == head_of_col[None, :],
                     a[:, None], 0.0).astype(F32)


def _bcast_mat(heads, oc):
    head_of_col = jnp.arange(heads * oc) // oc
    return (jnp.arange(heads)[:, None] == head_of_col[None, :]).astype(F32)


def _tc_pre(x, W, Asrc, Adst):
    DH = W.shape[1]

    def body(x_ref, w_ref, as_ref, ad_ref, h_ref, ts_ref, td_ref, gm_ref, mscr):
        xv = x_ref[...]
        hm = jnp.dot(xv, w_ref[...], preferred_element_type=F32)
        h_ref[...] = hm
        ts = jnp.dot(hm, as_ref[...], preferred_element_type=F32)
        td = jnp.dot(hm, ad_ref[...], preferred_element_type=F32)
        ts_ref[...] = ts
        td_ref[...] = td

        @pl.when(pl.program_id(0) == 0)
        def _():
            mscr[...] = jnp.full((1, 16), -1e30, F32)

        mscr[...] = jnp.maximum(mscr[...], jnp.max(ts, axis=0, keepdims=True))
        gm_ref[...] = mscr[...]

    return pl.pallas_call(
        body,
        grid=(TGRID,),
        in_specs=[
            pl.BlockSpec((RBLK, x.shape[1]), lambda i: (i, 0)),
            pl.BlockSpec(W.shape, lambda i: (0, 0)),
            pl.BlockSpec((DH, 16), lambda i: (0, 0)),
            pl.BlockSpec((DH, 16), lambda i: (0, 0)),
        ],
        out_specs=[
            pl.BlockSpec((RBLK, DH), lambda i: (i, 0)),
            pl.BlockSpec((RBLK, 16), lambda i: (i, 0)),
            pl.BlockSpec((RBLK, 16), lambda i: (i, 0)),
            pl.BlockSpec((1, 16), lambda i: (0, 0)),
        ],
        out_shape=[
            jax.ShapeDtypeStruct((NP, DH), F32),
            jax.ShapeDtypeStruct((NP, 16), F32),
            jax.ShapeDtypeStruct((NP, 16), F32),
            jax.ShapeDtypeStruct((1, 16), F32),
        ],
        scratch_shapes=[pltpu.VMEM((1, 16), F32)],
    )(x, W, Asrc, Adst)


def _tc_mid(acc, bias, B, W, Asrc, Adst, dprev, hprev):
    DH = W.shape[1]
    ACCP = acc.shape[2]

    def body(acc_ref, b_ref, bm_ref, w_ref, as_ref, ad_ref,
             x_ref, h_ref, ts_ref, td_ref, gm_ref, mscr):
        a = acc_ref[0] + acc_ref[1]
        num = a[:, :dprev]
        den = a[:, dprev:dprev + hprev]
        dexp = jnp.dot(den, bm_ref[...], preferred_element_type=F32)
        xq = num / (dexp + 1e-16) + b_ref[...]
        xv = jnp.where(xq > 0, xq, jnp.exp(xq) - 1.0)
        x_ref[...] = xv
        hm = jnp.dot(xv, w_ref[...], preferred_element_type=F32)
        h_ref[...] = hm
        ts = jnp.dot(hm, as_ref[...], preferred_element_type=F32)
        td = jnp.dot(hm, ad_ref[...], preferred_element_type=F32)
        ts_ref[...] = ts
        td_ref[...] = td

        @pl.when(pl.program_id(0) == 0)
        def _():
            mscr[...] = jnp.full((1, 16), -1e30, F32)

        mscr[...] = jnp.maximum(mscr[...], jnp.max(ts, axis=0, keepdims=True))
        gm_ref[...] = mscr[...]

    return pl.pallas_call(
        body,
        grid=(TGRID,),
        in_specs=[
            pl.BlockSpec((2, RBLK, ACCP), lambda i: (0, i, 0)),
            pl.BlockSpec((1, dprev), lambda i: (0, 0)),
            pl.BlockSpec((hprev, dprev), lambda i: (0, 0)),
            pl.BlockSpec(W.shape, lambda i: (0, 0)),
            pl.BlockSpec((DH, 16), lambda i: (0, 0)),
            pl.BlockSpec((DH, 16), lambda i: (0, 0)),
        ],
        out_specs=[
            pl.BlockSpec((RBLK, dprev), lambda i: (i, 0)),
            pl.BlockSpec((RBLK, DH), lambda i: (i, 0)),
            pl.BlockSpec((RBLK, 16), lambda i: (i, 0)),
            pl.BlockSpec((RBLK, 16), lambda i: (i, 0)),
            pl.BlockSpec((1, 16), lambda i: (0, 0)),
        ],
        out_shape=[
            jax.ShapeDtypeStruct((NP, dprev), F32),
            jax.ShapeDtypeStruct((NP, DH), F32),
            jax.ShapeDtypeStruct((NP, 16), F32),
            jax.ShapeDtypeStruct((NP, 16), F32),
            jax.ShapeDtypeStruct((1, 16), F32),
        ],
        scratch_shapes=[pltpu.VMEM((1, 16), F32)],
    )(acc, bias, B, W, Asrc, Adst)


def _tc_post(acc, bias, B, dprev, hprev):
    ACCP = acc.shape[2]

    def body(acc_ref, b_ref, bm_ref, x_ref):
        a = acc_ref[0] + acc_ref[1]
        num = a[:, :dprev]
        den = a[:, dprev:dprev + hprev]
        dexp = jnp.dot(den, bm_ref[...], preferred_element_type=F32)
        xq = num / (dexp + 1e-16) + b_ref[...]
        x_ref[...] = jnp.where(xq > 0, xq, jnp.exp(xq) - 1.0)

    return pl.pallas_call(
        body,
        grid=(TGRID,),
        in_specs=[
            pl.BlockSpec((2, RBLK, ACCP), lambda i: (0, i, 0)),
            pl.BlockSpec((1, dprev), lambda i: (0, 0)),
            pl.BlockSpec((hprev, dprev), lambda i: (0, 0)),
        ],
        out_specs=pl.BlockSpec((RBLK, dprev), lambda i: (i, 0)),
        out_shape=jax.ShapeDtypeStruct((NP, dprev), F32),
    )(acc, bias, B)


def _sc_edge(h, tsrc, tdst, gmax, srcv, dstv, D, H, chunk=16):
    ACC = D + 16
    EP = srcv.shape[0]
    EPW = EP // NWORK
    nchunk = EPW // chunk
    nmul = D // 16
    mesh = plsc.VectorSubcoreMesh(core_axis_name="c", subcore_axis_name="s")

    def body(h_hbm, ts_hbm, td_hbm, gm_hbm, src_hbm, dst_hbm, out_hbm,
             srcb, dstb, tsb, tdb, hb, msgb, gmv, zb, accs,
             sg1, sg2, sg3):
        cid = lax.axis_index("c")
        sid = lax.axis_index("s")
        wid = sid * NCORE + cid

        for r in range(16):
            for c in range(ACC // 16):
                zb[r, pl.ds(c * 16, 16)] = jnp.zeros((16,), F32)

        def zloop(j, carry):
            pltpu.sync_copy(zb, accs.at[pl.ds(sid * ROWS_PER_TILE + j * 16, 16)])
            return carry

        lax.fori_loop(0, ROWS_PER_TILE // 16, zloop, 0)

        pltpu.sync_copy(gm_hbm, gmv)
        base = wid * EPW
        pltpu.sync_copy(src_hbm.at[pl.ds(base, EPW)], srcb)
        pltpu.sync_copy(dst_hbm.at[pl.ds(base, EPW)], dstb)
        plsc.subcore_barrier()

        def chunk_body(t, carry):
            srciv = srcb[pl.ds(t * chunk, chunk)]
            dstiv = dstb[pl.ds(t * chunk, chunk)]
            c1 = pltpu.async_copy(ts_hbm.at[srciv], tsb, sg1)
            c2 = pltpu.async_copy(td_hbm.at[dstiv], tdb, sg2)
            c3 = pltpu.async_copy(h_hbm.at[srciv], hb, sg3)
            c1.wait()
            c2.wait()
            c3.wait()
            gm = gmv[...]
            for e in range(chunk):
                asv = tsb[e, :]
                adv = tdb[e, :]
                al = asv + adv
                al = jnp.maximum(al, 0.2 * al)
                sh = adv + gm
                sh = jnp.maximum(sh, 0.2 * sh)
                ev = jnp.exp(al - sh)
                msgb[e, pl.ds(D, 16)] = ev
                if H == 1:
                    for m in range(nmul):
                        msgb[e, pl.ds(m * 16, 16)] = hb[e, pl.ds(m * 16, 16)] * ev
                else:
                    for m in range(nmul):
                        evh = jnp.take_along_axis(
                            ev, jnp.full((16,), m % H, jnp.int32), axis=0,
                            mode="promise_in_bounds")
                        msgb[e, pl.ds(m * 16, 16)] = hb[e, pl.ds(m * 16, 16)] * evh
            pltpu.sync_copy(msgb, accs.at[dstiv], add=True)
            return carry

        lax.fori_loop(0, nchunk, chunk_body, 0)
        plsc.subcore_barrier()
        pltpu.sync_copy(accs.at[pl.ds(sid * ROWS_PER_TILE, ROWS_PER_TILE)],
                        out_hbm.at[cid, pl.ds(sid * ROWS_PER_TILE, ROWS_PER_TILE)])

    kfn = pl.kernel(
        body,
        out_type=jax.ShapeDtypeStruct((NCORE, NP, ACC), F32),
        mesh=mesh,
        compiler_params=pltpu.CompilerParams(needs_layout_passes=False,
                                             use_tc_tiling_on_sc=False),
        scratch_types=[
            pltpu.VMEM((EPW,), jnp.int32),
            pltpu.VMEM((EPW,), jnp.int32),
            pltpu.VMEM((chunk, 16), F32),
            pltpu.VMEM((chunk, 16), F32),
            pltpu.VMEM((chunk, D), F32),
            pltpu.VMEM((chunk, ACC), F32),
            pltpu.VMEM((16,), F32),
            pltpu.VMEM((16, ACC), F32),
            pltpu.VMEM_SHARED((NP, ACC), F32),
            pltpu.SemaphoreType.DMA,
            pltpu.SemaphoreType.DMA,
            pltpu.SemaphoreType.DMA,
        ],
    )
    return kfn(h, tsrc, tdst, gmax, srcv, dstv)


def kernel(x, edge_index, params):
    layers = params["layers"]
    cfg = [(8, 16)] * 4 + [(1, 32)]

    src = edge_index[0].astype(jnp.int32)
    dst = edge_index[1].astype(jnp.int32)
    loop = jnp.arange(NNODE, dtype=jnp.int32)
    etot = NEDGE + NNODE
    ep = ((etot + NWORK * 16 - 1) // (NWORK * 16)) * NWORK * 16
    pad = ep - etot
    srcv = jnp.concatenate([src, loop, jnp.zeros((pad,), jnp.int32)])
    dstv = jnp.concatenate([dst, loop, jnp.full((pad,), NNODE, jnp.int32)])
    xp = jnp.pad(x, ((0, NP - NNODE), (0, 0)))

    p0 = layers[0]
    h, ts, td, gm = _tc_pre(xp, p0["W"],
                            _att_mat(p0["att_src"], 8, 16),
                            _att_mat(p0["att_dst"], 8, 16))
    acc = _sc_edge(h, ts, td, gm.reshape(16), srcv, dstv, D=128, H=8)

    B8 = _bcast_mat(8, 16)
    reps = []
    for li in range(1, 5):
        p = layers[li]
        heads, oc = cfg[li]
        prev_bias = layers[li - 1]["bias"].reshape(1, -1).astype(F32)
        xl, h, ts, td, gm = _tc_mid(acc, prev_bias, B8, p["W"],
                                    _att_mat(p["att_src"], heads, oc),
                                    _att_mat(p["att_dst"], heads, oc),
                                    dprev=128, hprev=8)
        reps.append(xl[:NNODE])
        acc = _sc_edge(h, ts, td, gm.reshape(16), srcv, dstv,
                       D=heads * oc, H=heads)

    out = _tc_post(acc, layers[4]["bias"].reshape(1, -1).astype(F32),
                   jnp.ones((1, 32), F32), dprev=32, hprev=1)
    outn = out[:NNODE]
    reps.append(outn)
    return (outn, reps)


# speedup vs baseline: 105.5788x; 1.8780x over previous
import functools

import jax
import jax.numpy as jnp
from jax import lax
from jax.experimental import pallas as pl
from jax.experimental.pallas import tpu as pltpu
from jax.experimental.pallas import tpu_sc as plsc

NNODE = 10000
NP = 10240
NEDGE = 320000
NCORE = 2
NSUB = 16
NWORK = NCORE * NSUB
ROWS_PER_TILE = NP // NSUB
F32 = jnp.float32

RBLK = 1024
TGRID = NP // RBLK


def _att_mat(att, heads, oc):
    a = att.reshape(heads * oc)
    head_of_row = jnp.arange(heads * oc) // oc
    head_of_col = jnp.arange(16) % heads
    return jnp.where(head_of_row[:, None] == head_of_col[None, :],
                     a[:, None], 0.0).astype(F32)


def _bcast_mat(heads, oc):
    head_of_col = jnp.arange(heads * oc) // oc
    return (jnp.arange(heads)[:, None] == head_of_col[None, :]).astype(F32)


def _tc_pre(x, W, Asrc, Adst):
    DH = W.shape[1]

    def body(x_ref, w_ref, as_ref, ad_ref, h_ref, ts_ref, td_ref, gm_ref, mscr):
        xv = x_ref[...]
        hm = jnp.dot(xv, w_ref[...], preferred_element_type=F32)
        h_ref[...] = hm
        ts = jnp.dot(hm, as_ref[...], preferred_element_type=F32)
        td = jnp.dot(hm, ad_ref[...], preferred_element_type=F32)
        ts_ref[...] = ts
        td_ref[...] = td

        @pl.when(pl.program_id(0) == 0)
        def _():
            mscr[...] = jnp.full((1, 16), -1e30, F32)

        mscr[...] = jnp.maximum(mscr[...], jnp.max(ts, axis=0, keepdims=True))
        gm_ref[...] = mscr[...]

    return pl.pallas_call(
        body,
        grid=(TGRID,),
        in_specs=[
            pl.BlockSpec((RBLK, x.shape[1]), lambda i: (i, 0)),
            pl.BlockSpec(W.shape, lambda i: (0, 0)),
            pl.BlockSpec((DH, 16), lambda i: (0, 0)),
            pl.BlockSpec((DH, 16), lambda i: (0, 0)),
        ],
        out_specs=[
            pl.BlockSpec((RBLK, DH), lambda i: (i, 0)),
            pl.BlockSpec((RBLK, 16), lambda i: (i, 0)),
            pl.BlockSpec((RBLK, 16), lambda i: (i, 0)),
            pl.BlockSpec((1, 16), lambda i: (0, 0)),
        ],
        out_shape=[
            jax.ShapeDtypeStruct((NP, DH), F32),
            jax.ShapeDtypeStruct((NP, 16), F32),
            jax.ShapeDtypeStruct((NP, 16), F32),
            jax.ShapeDtypeStruct((1, 16), F32),
        ],
        scratch_shapes=[pltpu.VMEM((1, 16), F32)],
    )(x, W, Asrc, Adst)


def _tc_mid(acc, bias, B, W, Asrc, Adst, dprev, hprev):
    DH = W.shape[1]
    ACCP = acc.shape[2]

    def body(acc_ref, b_ref, bm_ref, w_ref, as_ref, ad_ref,
             x_ref, h_ref, ts_ref, td_ref, gm_ref, mscr):
        a = acc_ref[0] + acc_ref[1]
        num = a[:, :dprev]
        den = a[:, dprev:dprev + hprev]
        dexp = jnp.dot(den, bm_ref[...], preferred_element_type=F32)
        xq = num / (dexp + 1e-16) + b_ref[...]
        xv = jnp.where(xq > 0, xq, jnp.exp(xq) - 1.0)
        x_ref[...] = xv
        hm = jnp.dot(xv, w_ref[...], preferred_element_type=F32)
        h_ref[...] = hm
        ts = jnp.dot(hm, as_ref[...], preferred_element_type=F32)
        td = jnp.dot(hm, ad_ref[...], preferred_element_type=F32)
        ts_ref[...] = ts
        td_ref[...] = td

        @pl.when(pl.program_id(0) == 0)
        def _():
            mscr[...] = jnp.full((1, 16), -1e30, F32)

        mscr[...] = jnp.maximum(mscr[...], jnp.max(ts, axis=0, keepdims=True))
        gm_ref[...] = mscr[...]

    return pl.pallas_call(
        body,
        grid=(TGRID,),
        in_specs=[
            pl.BlockSpec((2, RBLK, ACCP), lambda i: (0, i, 0)),
            pl.BlockSpec((1, dprev), lambda i: (0, 0)),
            pl.BlockSpec((hprev, dprev), lambda i: (0, 0)),
            pl.BlockSpec(W.shape, lambda i: (0, 0)),
            pl.BlockSpec((DH, 16), lambda i: (0, 0)),
            pl.BlockSpec((DH, 16), lambda i: (0, 0)),
        ],
        out_specs=[
            pl.BlockSpec((RBLK, dprev), lambda i: (i, 0)),
            pl.BlockSpec((RBLK, DH), lambda i: (i, 0)),
            pl.BlockSpec((RBLK, 16), lambda i: (i, 0)),
            pl.BlockSpec((RBLK, 16), lambda i: (i, 0)),
            pl.BlockSpec((1, 16), lambda i: (0, 0)),
        ],
        out_shape=[
            jax.ShapeDtypeStruct((NP, dprev), F32),
            jax.ShapeDtypeStruct((NP, DH), F32),
            jax.ShapeDtypeStruct((NP, 16), F32),
            jax.ShapeDtypeStruct((NP, 16), F32),
            jax.ShapeDtypeStruct((1, 16), F32),
        ],
        scratch_shapes=[pltpu.VMEM((1, 16), F32)],
    )(acc, bias, B, W, Asrc, Adst)


def _tc_post(acc, bias, B, dprev, hprev):
    ACCP = acc.shape[2]

    def body(acc_ref, b_ref, bm_ref, x_ref):
        a = acc_ref[0] + acc_ref[1]
        num = a[:, :dprev]
        den = a[:, dprev:dprev + hprev]
        dexp = jnp.dot(den, bm_ref[...], preferred_element_type=F32)
        xq = num / (dexp + 1e-16) + b_ref[...]
        x_ref[...] = jnp.where(xq > 0, xq, jnp.exp(xq) - 1.0)

    return pl.pallas_call(
        body,
        grid=(TGRID,),
        in_specs=[
            pl.BlockSpec((2, RBLK, ACCP), lambda i: (0, i, 0)),
            pl.BlockSpec((1, dprev), lambda i: (0, 0)),
            pl.BlockSpec((hprev, dprev), lambda i: (0, 0)),
        ],
        out_specs=pl.BlockSpec((RBLK, dprev), lambda i: (i, 0)),
        out_shape=jax.ShapeDtypeStruct((NP, dprev), F32),
    )(acc, bias, B)


def _sc_edge(h, tsrc, tdst, gmax, srcv, dstv, D, H, chunk=16):
    ACC = D + 16
    EP = srcv.shape[0]
    EPW = EP // NWORK
    nchunk = EPW // chunk
    nmul = D // 16
    mesh = plsc.VectorSubcoreMesh(core_axis_name="c", subcore_axis_name="s")

    def body(h_hbm, ts_hbm, td_hbm, gm_hbm, src_hbm, dst_hbm, out_hbm,
             srcb, dstb, tsb, tdb, hb, msgb, gmv, zb, accs,
             st0, st1, sd0, sd1, sh0, sh1):
        sems_ts = (st0, st1)
        sems_td = (sd0, sd1)
        sems_h = (sh0, sh1)
        cid = lax.axis_index("c")
        sid = lax.axis_index("s")
        wid = sid * NCORE + cid

        for r in range(16):
            for c in range(ACC // 16):
                zb[r, pl.ds(c * 16, 16)] = jnp.zeros((16,), F32)

        def zloop(j, carry):
            pltpu.sync_copy(zb, accs.at[pl.ds(sid * ROWS_PER_TILE + j * 16, 16)])
            return carry

        lax.fori_loop(0, ROWS_PER_TILE // 16, zloop, 0)

        pltpu.sync_copy(gm_hbm, gmv)
        base = wid * EPW
        pltpu.sync_copy(src_hbm.at[pl.ds(base, EPW)], srcb.at[pl.ds(0, EPW)])
        pltpu.sync_copy(dst_hbm.at[pl.ds(base, EPW)], dstb.at[pl.ds(0, EPW)])
        for k in range(2 * chunk // 16):
            srcb[pl.ds(EPW + k * 16, 16)] = jnp.zeros((16,), jnp.int32)
            dstb[pl.ds(EPW + k * 16, 16)] = jnp.zeros((16,), jnp.int32)
        plsc.subcore_barrier()

        def gathers(t, b):
            sidx = srcb.at[pl.ds(t * chunk, chunk)]
            didx = dstb.at[pl.ds(t * chunk, chunk)]
            return (pltpu.make_async_copy(ts_hbm.at[sidx], tsb.at[b], sems_ts[b]),
                    pltpu.make_async_copy(td_hbm.at[didx], tdb.at[b], sems_td[b]),
                    pltpu.make_async_copy(h_hbm.at[sidx], hb.at[b], sems_h[b]))

        def fire(t, b):
            for c in gathers(t, b):
                c.start()

        def drain(t, b):
            for c in gathers(t, b):
                c.wait()

        for b in range(2):
            fire(b, b)

        def pair_body(i2, carry):
            gm = gmv[...]
            for b in range(2):
                t = i2 * 2 + b
                drain(t, b)
                fire(t + 2, b)
                for e in range(chunk):
                    asv = tsb[b, e, :]
                    adv = tdb[b, e, :]
                    al = asv + adv
                    al = jnp.maximum(al, 0.2 * al)
                    sh = adv + gm
                    sh = jnp.maximum(sh, 0.2 * sh)
                    ev = jnp.exp(al - sh)
                    msgb[e, pl.ds(D, 16)] = ev
                    if H == 1:
                        for m in range(nmul):
                            msgb[e, pl.ds(m * 16, 16)] = (
                                hb[b, e, pl.ds(m * 16, 16)] * ev)
                    else:
                        for m in range(nmul):
                            evh = jnp.take_along_axis(
                                ev, jnp.full((16,), m % H, jnp.int32), axis=0,
                                mode="promise_in_bounds")
                            msgb[e, pl.ds(m * 16, 16)] = (
                                hb[b, e, pl.ds(m * 16, 16)] * evh)
                dstiv = dstb[pl.ds(t * chunk, chunk)]
                pltpu.sync_copy(msgb, accs.at[dstiv], add=True)
            return carry

        lax.fori_loop(0, nchunk // 2, pair_body, 0)
        for b in range(2):
            drain(0, b)
        plsc.subcore_barrier()
        pltpu.sync_copy(accs.at[pl.ds(sid * ROWS_PER_TILE, ROWS_PER_TILE)],
                        out_hbm.at[cid, pl.ds(sid * ROWS_PER_TILE, ROWS_PER_TILE)])

    kfn = pl.kernel(
        body,
        out_type=jax.ShapeDtypeStruct((NCORE, NP, ACC), F32),
        mesh=mesh,
        compiler_params=pltpu.CompilerParams(needs_layout_passes=False,
                                             use_tc_tiling_on_sc=False),
        scratch_types=[
            pltpu.VMEM((EPW + 2 * chunk,), jnp.int32),
            pltpu.VMEM((EPW + 2 * chunk,), jnp.int32),
            pltpu.VMEM((2, chunk, 16), F32),
            pltpu.VMEM((2, chunk, 16), F32),
            pltpu.VMEM((2, chunk, D), F32),
            pltpu.VMEM((chunk, ACC), F32),
            pltpu.VMEM((16,), F32),
            pltpu.VMEM((16, ACC), F32),
            pltpu.VMEM_SHARED((NP, ACC), F32),
            pltpu.SemaphoreType.DMA,
            pltpu.SemaphoreType.DMA,
            pltpu.SemaphoreType.DMA,
            pltpu.SemaphoreType.DMA,
            pltpu.SemaphoreType.DMA,
            pltpu.SemaphoreType.DMA,
        ],
    )
    return kfn(h, tsrc, tdst, gmax, srcv, dstv)


def kernel(x, edge_index, params):
    layers = params["layers"]
    cfg = [(8, 16)] * 4 + [(1, 32)]

    src = edge_index[0].astype(jnp.int32)
    dst = edge_index[1].astype(jnp.int32)
    loop = jnp.arange(NNODE, dtype=jnp.int32)
    etot = NEDGE + NNODE
    ep = ((etot + NWORK * 32 - 1) // (NWORK * 32)) * NWORK * 32
    pad = ep - etot
    srcv = jnp.concatenate([src, loop, jnp.zeros((pad,), jnp.int32)])
    dstv = jnp.concatenate([dst, loop, jnp.full((pad,), NNODE, jnp.int32)])
    xp = jnp.pad(x, ((0, NP - NNODE), (0, 0)))

    p0 = layers[0]
    h, ts, td, gm = _tc_pre(xp, p0["W"],
                            _att_mat(p0["att_src"], 8, 16),
                            _att_mat(p0["att_dst"], 8, 16))
    acc = _sc_edge(h, ts, td, gm.reshape(16), srcv, dstv, D=128, H=8)

    B8 = _bcast_mat(8, 16)
    reps = []
    for li in range(1, 5):
        p = layers[li]
        heads, oc = cfg[li]
        prev_bias = layers[li - 1]["bias"].reshape(1, -1).astype(F32)
        xl, h, ts, td, gm = _tc_mid(acc, prev_bias, B8, p["W"],
                                    _att_mat(p["att_src"], heads, oc),
                                    _att_mat(p["att_dst"], heads, oc),
                                    dprev=128, hprev=8)
        reps.append(xl[:NNODE])
        acc = _sc_edge(h, ts, td, gm.reshape(16), srcv, dstv,
                       D=heads * oc, H=heads)

    out = _tc_post(acc, layers[4]["bias"].reshape(1, -1).astype(F32),
                   jnp.ones((1, 32), F32), dprev=32, hprev=1)
    outn = out[:NNODE]
    reps.append(outn)
    return (outn, reps)
